# baseline (speedup 1.0000x reference)
import functools

import jax
import jax.numpy as jnp
from jax import lax
from jax.experimental import pallas as pl
from jax.experimental.pallas import tpu as pltpu
from jax.experimental.pallas import tpu_sc as plsc

VOCAB = 32
D = 256
NTOK = 64 * 2048
NC = 2
NS = 16
NW = NC * NS
B_PER_W = NTOK // NW
CHUNK = 128
NCHUNK = B_PER_W // CHUNK
TW = CHUNK * D
LANES = 16
SCALE = 16.0

_mesh = plsc.VectorSubcoreMesh(core_axis_name="c", subcore_axis_name="s")


@functools.partial(
    pl.kernel,
    out_type=jax.ShapeDtypeStruct((NTOK, D), jnp.float32),
    mesh=_mesh,
    scratch_types=dict(
        ltab=pltpu.VMEM((VOCAB, D), jnp.float32),
        idx_v=pltpu.VMEM((B_PER_W,), jnp.int32),
        bufs=pltpu.VMEM((2, CHUNK, D), jnp.float32),
        s0=pltpu.SemaphoreType.DMA,
        s1=pltpu.SemaphoreType.DMA,
    ),
    compiler_params=pltpu.CompilerParams(needs_layout_passes=False),
)
def _emb_kernel(x_hbm, tabf_hbm, out_hbm, ltab, idx_v, bufs, s0, s1):
    c = lax.axis_index("c")
    s = lax.axis_index("s")
    wid = s * NC + c
    base = wid * B_PER_W

    pltpu.sync_copy(tabf_hbm, ltab)

    def scale_body(r, carry):
        def col_body(j, carry2):
            v = ltab[r, pl.ds(j * LANES, LANES)]
            ltab[r, pl.ds(j * LANES, LANES)] = v * SCALE
            return carry2
        return lax.fori_loop(0, D // LANES, col_body, carry)
    lax.fori_loop(0, VOCAB, scale_body, 0)

    pltpu.sync_copy(x_hbm.at[pl.ds(base, B_PER_W)], idx_v)

    lane = lax.broadcasted_iota(jnp.int32, (LANES,), 0)

    ssems = (s0, s1)

    def mk_scatter(ci, par):
        return pltpu.make_async_copy(
            bufs.at[par],
            out_hbm.at[pl.ds(base + ci * CHUNK, CHUNK)],
            ssems[par],
        )

    def compute_chunk(ci, par):
        def group(g, carry):
            rows = idx_v[pl.ds(ci * CHUNK + g * LANES, LANES)]
            tokpos = lane + g * LANES
            buf = bufs.at[par]
            for cc in range(D):
                col = jnp.full((LANES,), cc, jnp.int32)
                v = plsc.load_gather(ltab, [rows, col])
                plsc.store_scatter(buf, [tokpos, col], v)
            return carry
        lax.fori_loop(0, CHUNK // LANES, group, 0)

    def outer(i, carry):
        for par in (0, 1):
            ci = i * 2 + par

            @pl.when(i > 0)
            def _wait_prev():
                mk_scatter(ci - 2, par).wait()

            compute_chunk(ci, par)
            mk_scatter(ci, par).start()
        return carry
    lax.fori_loop(0, NCHUNK // 2, outer, 0)

    for par in (0, 1):
        mk_scatter(NCHUNK - 2 + par, par).wait()


def kernel(x, table):
    b, sq = x.shape
    out = _emb_kernel(x.reshape(-1).astype(jnp.int32), table)
    return out.reshape(b, sq, D)


# speedup vs baseline: 4.7579x; 4.7579x over previous
import functools

import jax
import jax.numpy as jnp
from jax import lax
from jax.experimental import pallas as pl
from jax.experimental.pallas import tpu as pltpu
from jax.experimental.pallas import tpu_sc as plsc

VOCAB = 32
D = 256
NTOK = 64 * 2048
NC = 2
NS = 16
NW = NC * NS
B_PER_W = NTOK // NW
CHUNK = 128
NCHUNK = B_PER_W // CHUNK
TW = CHUNK * D
LANES = 16
SCALE = 16.0

_mesh = plsc.VectorSubcoreMesh(core_axis_name="c", subcore_axis_name="s")


@functools.partial(
    pl.kernel,
    out_type=jax.ShapeDtypeStruct((NTOK, D), jnp.float32),
    mesh=_mesh,
    scratch_types=dict(
        ltab=pltpu.VMEM((VOCAB, D), jnp.float32),
        idx_v=pltpu.VMEM((B_PER_W,), jnp.int32),
        bufs=pltpu.VMEM((2, CHUNK, D), jnp.float32),
        s0=pltpu.SemaphoreType.DMA,
        s1=pltpu.SemaphoreType.DMA,
    ),
    compiler_params=pltpu.CompilerParams(needs_layout_passes=False),
)
def _emb_kernel(x_hbm, tabf_hbm, out_hbm, ltab, idx_v, bufs, s0, s1):
    c = lax.axis_index("c")
    s = lax.axis_index("s")
    wid = s * NC + c
    base = wid * B_PER_W

    pltpu.sync_copy(tabf_hbm, ltab)

    def scale_body(r, carry):
        def col_body(j, carry2):
            v = ltab[r, pl.ds(j * LANES, LANES)]
            ltab[r, pl.ds(j * LANES, LANES)] = v * SCALE
            return carry2
        return lax.fori_loop(0, D // LANES, col_body, carry)
    lax.fori_loop(0, VOCAB, scale_body, 0)

    pltpu.sync_copy(x_hbm.at[pl.ds(base, B_PER_W)], idx_v)

    lane = lax.broadcasted_iota(jnp.int32, (LANES,), 0)

    ssems = (s0, s1)

    def mk_scatter(ci, par):
        return pltpu.make_async_copy(
            bufs.at[par],
            out_hbm.at[pl.ds(base + ci * CHUNK, CHUNK)],
            ssems[par],
        )

    def compute_chunk(ci, par):
        def group(g, carry):
            rows = idx_v[pl.ds(ci * CHUNK + g * LANES, LANES)]
            buf = bufs.at[par]
            for t in range(LANES):
                r = rows[t]
                tp = g * LANES + t
                for j in range(D // LANES):
                    buf[tp, pl.ds(j * LANES, LANES)] = ltab[r, pl.ds(j * LANES, LANES)]
            return carry
        lax.fori_loop(0, CHUNK // LANES, group, 0)

    def outer(i, carry):
        for par in (0, 1):
            ci = i * 2 + par

            @pl.when(i > 0)
            def _wait_prev():
                mk_scatter(ci - 2, par).wait()

            compute_chunk(ci, par)
            mk_scatter(ci, par).start()
        return carry
    lax.fori_loop(0, NCHUNK // 2, outer, 0)

    for par in (0, 1):
        mk_scatter(NCHUNK - 2 + par, par).wait()


def kernel(x, table):
    b, sq = x.shape
    out = _emb_kernel(x.reshape(-1).astype(jnp.int32), table)
    return out.reshape(b, sq, D)


# speedup vs baseline: 18.4355x; 3.8747x over previous
import functools

import jax
import jax.numpy as jnp
from jax import lax
from jax.experimental import pallas as pl
from jax.experimental.pallas import tpu as pltpu
from jax.experimental.pallas import tpu_sc as plsc

VOCAB = 32
D = 256
NTOK = 64 * 2048
NC = 2
NS = 16
NW = NC * NS
B_PER_W = NTOK // NW
CHUNK = 128
NCHUNK = B_PER_W // CHUNK
TW = CHUNK * D
LANES = 16
SCALE = 16.0

_mesh = plsc.VectorSubcoreMesh(core_axis_name="c", subcore_axis_name="s")


@functools.partial(
    pl.kernel,
    out_type=jax.ShapeDtypeStruct((NTOK, D), jnp.float32),
    mesh=_mesh,
    scratch_types=dict(
        ltab=pltpu.VMEM((VOCAB, D), jnp.float32),
        idx_v=pltpu.VMEM((B_PER_W,), jnp.int32),
        s0=pltpu.SemaphoreType.DMA,
    ),
    compiler_params=pltpu.CompilerParams(needs_layout_passes=False),
)
def _emb_kernel(x_hbm, tabf_hbm, out_hbm, ltab, idx_v, s0):
    c = lax.axis_index("c")
    s = lax.axis_index("s")
    wid = s * NC + c
    base = wid * B_PER_W

    pltpu.sync_copy(tabf_hbm, ltab)

    def scale_body(r, carry):
        def col_body(j, carry2):
            v = ltab[r, pl.ds(j * LANES, LANES)]
            ltab[r, pl.ds(j * LANES, LANES)] = v * SCALE
            return carry2
        return lax.fori_loop(0, D // LANES, col_body, carry)
    lax.fori_loop(0, VOCAB, scale_body, 0)

    pltpu.sync_copy(x_hbm.at[pl.ds(base, B_PER_W)], idx_v)

    def group(gi, carry):
        rows = idx_v[pl.ds(gi * LANES, LANES)]
        for t in range(LANES):
            r = rows[t]
            pltpu.make_async_copy(
                ltab.at[r],
                out_hbm.at[base + gi * LANES + t],
                s0,
            ).start()
        return carry
    lax.fori_loop(0, B_PER_W // LANES, group, 0)

    pltpu.make_async_copy(
        out_hbm.at[pl.ds(0, B_PER_W)],
        out_hbm.at[pl.ds(base, B_PER_W)],
        s0,
    ).wait()


def kernel(x, table):
    b, sq = x.shape
    out = _emb_kernel(x.reshape(-1).astype(jnp.int32), table)
    return out.reshape(b, sq, D)
